# Initial kernel scaffold; baseline (speedup 1.0000x reference)
import jax
import jax.numpy as jnp
from jax.experimental import pallas as pl


def kernel(points):
    raise NotImplementedError("write your pallas kernel here")


# speedup vs baseline: 1.7783x; 1.7783x over previous
import jax
import jax.numpy as jnp
from jax.experimental import pallas as pl

_NUM_CLASS_CH = 20


def _nms_block(x_ref, o_ref):
    x = x_ref[0]
    probs = jnp.max(x[:_NUM_CLASS_CH], axis=0)
    H, W = probs.shape
    zrow = jnp.zeros((1, W), probs.dtype)
    zcol = jnp.zeros((H, 1), probs.dtype)

    def shl(a):
        return jnp.concatenate([zcol, a[:, :-1]], axis=1)

    def shr(a):
        return jnp.concatenate([a[:, 1:], zcol], axis=1)

    up = jnp.concatenate([zrow, probs[:-1]], axis=0)
    down = jnp.concatenate([probs[1:], zrow], axis=0)

    strict = jnp.maximum(jnp.maximum(shl(up), up),
                         jnp.maximum(shr(up), shl(probs)))
    nonstrict = jnp.maximum(jnp.maximum(shr(probs), shl(down)),
                            jnp.maximum(down, shr(down)))
    mask = ((probs > strict) & (probs >= nonstrict)).astype(x.dtype)
    o_ref[0] = x * mask[None, :, :]


def kernel(points):
    B, C, H, W = points.shape
    return pl.pallas_call(
        _nms_block,
        grid=(B,),
        in_specs=[pl.BlockSpec((1, C, H, W), lambda b: (b, 0, 0, 0))],
        out_specs=pl.BlockSpec((1, C, H, W), lambda b: (b, 0, 0, 0)),
        out_shape=jax.ShapeDtypeStruct(points.shape, points.dtype),
    )(points)


# speedup vs baseline: 1.8576x; 1.0446x over previous
import jax
import jax.numpy as jnp
from jax.experimental import pallas as pl

_NUM_CLASS_CH = 20


def _nms_block(x_ref, o_ref):
    x = x_ref[...]
    probs = jnp.max(x[:, :_NUM_CLASS_CH], axis=1)
    Bb, H, W = probs.shape
    zrow = jnp.zeros((Bb, 1, W), probs.dtype)
    zcol = jnp.zeros((Bb, H, 1), probs.dtype)

    def shl(a):
        return jnp.concatenate([zcol, a[:, :, :-1]], axis=2)

    def shr(a):
        return jnp.concatenate([a[:, :, 1:], zcol], axis=2)

    up = jnp.concatenate([zrow, probs[:, :-1]], axis=1)
    down = jnp.concatenate([probs[:, 1:], zrow], axis=1)

    strict = jnp.maximum(jnp.maximum(shl(up), up),
                         jnp.maximum(shr(up), shl(probs)))
    nonstrict = jnp.maximum(jnp.maximum(shr(probs), shl(down)),
                            jnp.maximum(down, shr(down)))
    mask = ((probs > strict) & (probs >= nonstrict)).astype(x.dtype)
    o_ref[...] = x * mask[:, None, :, :]


def kernel(points):
    B, C, H, W = points.shape
    return pl.pallas_call(
        _nms_block,
        grid=(B // 2,),
        in_specs=[pl.BlockSpec((2, C, H, W), lambda b: (b, 0, 0, 0))],
        out_specs=pl.BlockSpec((2, C, H, W), lambda b: (b, 0, 0, 0)),
        out_shape=jax.ShapeDtypeStruct(points.shape, points.dtype),
    )(points)
